# Initial kernel scaffold; baseline (speedup 1.0000x reference)
import jax
import jax.numpy as jnp
from jax.experimental import pallas as pl


def kernel(white_input, black_input, W_white, W_black, w1, b1, w2, b2, w3, b3):
    raise NotImplementedError("write your pallas kernel here")


# speedup vs baseline: 5.8431x; 5.8431x over previous
import functools

import jax
import jax.numpy as jnp
from jax import lax
from jax.experimental import pallas as pl
from jax.experimental.pallas import tpu as pltpu
from jax.experimental.pallas import tpu_sc as plsc

B, L, V, H = 16384, 32, 40960, 256
NC, NS = 2, 16
NW = NC * NS
CHUNK = 128
SUB = B // NW
NCHUNK = SUB // CHUNK
HV = H // 16


def _embed_bags(wT, bT, W_white, W_black):
    mesh = plsc.VectorSubcoreMesh(core_axis_name="c", subcore_axis_name="s",
                                  num_cores=NC, num_subcores=NS)

    @functools.partial(
        pl.kernel,
        out_type=(
            jax.ShapeDtypeStruct((B, H), jnp.float32),
            jax.ShapeDtypeStruct((B, H), jnp.float32),
        ),
        mesh=mesh,
        scratch_types=[
            pltpu.VMEM((L, CHUNK), jnp.int32),
            pltpu.VMEM((L, CHUNK), jnp.int32),
            pltpu.VMEM((CHUNK, H), jnp.float32),
            pltpu.VMEM((CHUNK, H), jnp.float32),
            pltpu.VMEM((CHUNK, H), jnp.float32),
            pltpu.SemaphoreType.DMA,
            pltpu.SemaphoreType.DMA,
        ],
    )
    def body(wT_h, bT_h, Ww_h, Wb_h, ow_h, ob_h,
             idxw, idxb, r0, r1, acc, sg0, sg1):
        wid = lax.axis_index("s") * NC + lax.axis_index("c")

        def acc_set(buf):
            @pl.loop(0, CHUNK)
            def _rows(r):
                for j in range(HV):
                    acc[r, pl.ds(16 * j, 16)] = buf[r, pl.ds(16 * j, 16)]

        def acc_add(buf):
            @pl.loop(0, CHUNK)
            def _rows(r):
                for j in range(HV):
                    plsc.addupdate(acc.at[r, pl.ds(16 * j, 16)],
                                   buf[r, pl.ds(16 * j, 16)])

        def do_table(W_h, idxv, o_h, base):
            pltpu.async_copy(W_h.at[idxv.at[0]], r0, sg0)
            pltpu.async_copy(W_h.at[idxv.at[1]], r1, sg1)
            pltpu.make_async_copy(W_h.at[idxv.at[0]], r0, sg0).wait()
            acc_set(r0)
            pltpu.async_copy(W_h.at[idxv.at[2]], r0, sg0)
            pltpu.make_async_copy(W_h.at[idxv.at[1]], r1, sg1).wait()
            acc_add(r1)
            pltpu.async_copy(W_h.at[idxv.at[3]], r1, sg1)

            @pl.loop(1, L // 2 - 1)
            def _go(g):
                l0 = 2 * g
                pltpu.make_async_copy(W_h.at[idxv.at[0]], r0, sg0).wait()
                acc_add(r0)
                pltpu.async_copy(W_h.at[idxv.at[l0 + 2]], r0, sg0)
                pltpu.make_async_copy(W_h.at[idxv.at[0]], r1, sg1).wait()
                acc_add(r1)
                pltpu.async_copy(W_h.at[idxv.at[l0 + 3]], r1, sg1)

            pltpu.make_async_copy(W_h.at[idxv.at[0]], r0, sg0).wait()
            acc_add(r0)
            pltpu.make_async_copy(W_h.at[idxv.at[0]], r1, sg1).wait()
            acc_add(r1)
            pltpu.sync_copy(acc, o_h.at[pl.ds(base, CHUNK)])

        @pl.loop(0, NCHUNK)
        def _chunks(c):
            base = wid * SUB + c * CHUNK
            pltpu.sync_copy(wT_h.at[:, pl.ds(base, CHUNK)], idxw)
            pltpu.sync_copy(bT_h.at[:, pl.ds(base, CHUNK)], idxb)
            do_table(Ww_h, idxw, ow_h, base)
            do_table(Wb_h, idxb, ob_h, base)

    return body(wT, bT, W_white, W_black)


BM = 2048


def _mlp_body(wv, bv, w1a, w1b, b1, w2t, b2, w3r, b3, o):
    f32 = jnp.float32
    hi = jax.lax.Precision.HIGHEST
    h = jnp.dot(wv[...], w1a[...], precision=hi, preferred_element_type=f32)
    h += jnp.dot(bv[...], w1b[...], precision=hi, preferred_element_type=f32)
    h = jnp.maximum(h + b1[...], 0.0)
    h = jnp.dot(h, w2t[...], precision=hi, preferred_element_type=f32)
    h = jnp.maximum(h + b2[...], 0.0)
    o[...] = jnp.sum(h * w3r[...], axis=1) + b3[0, 0]


def _mlp(white_vec, black_vec, w1, b1, w2, b2, w3, b3):
    w1a = w1[:, :H].T
    w1b = w1[:, H:].T
    grid = (B // BM,)
    full = lambda shape: pl.BlockSpec(shape, lambda i: (0, 0))
    return pl.pallas_call(
        _mlp_body,
        grid=grid,
        in_specs=[
            pl.BlockSpec((BM, H), lambda i: (i, 0)),
            pl.BlockSpec((BM, H), lambda i: (i, 0)),
            full((H, 32)),
            full((H, 32)),
            full((1, 32)),
            full((32, 32)),
            full((1, 32)),
            full((1, 32)),
            full((1, 1)),
        ],
        out_specs=pl.BlockSpec((BM,), lambda i: (i,)),
        out_shape=jax.ShapeDtypeStruct((B,), jnp.float32),
    )(white_vec, black_vec, w1a, w1b, b1.reshape(1, 32), w2.T,
      b2.reshape(1, 32), w3.reshape(1, 32), b3.reshape(1, 1))


def kernel(white_input, black_input, W_white, W_black, w1, b1, w2, b2, w3, b3):
    wT = white_input.T.astype(jnp.int32)
    bT = black_input.T.astype(jnp.int32)
    white_vec, black_vec = _embed_bags(wT, bT, W_white, W_black)
    return _mlp(white_vec, black_vec, w1, b1, w2, b2, w3, b3)
